# speedup vs baseline: 1.9886x; 1.9886x over previous
import functools

import jax
import jax.numpy as jnp
from jax import lax
from jax.experimental import pallas as pl
from jax.experimental.pallas import tpu as pltpu
from jax.experimental.pallas import tpu_sc as plsc

D = 1024
TOTAL = 4 * 8192
NW = 32
B_PER_W = TOTAL // NW
CHUNK = 32
N_CHUNKS = B_PER_W // CHUNK


def _make_gather():
    mesh = plsc.VectorSubcoreMesh(core_axis_name="c", subcore_axis_name="s")

    @functools.partial(
        pl.kernel,
        mesh=mesh,
        out_type=jax.ShapeDtypeStruct((TOTAL, D), jnp.float32),
        scratch_types=[
            pltpu.VMEM((B_PER_W,), jnp.int32),
            pltpu.VMEM((CHUNK, D), jnp.float32),
            pltpu.SemaphoreType.DMA,
        ],
    )
    def gather_kernel(idx_hbm, table_hbm, out_hbm, idx_v, rows_v, sem):
        wid = lax.axis_index("s") * 2 + lax.axis_index("c")
        base = wid * B_PER_W
        pltpu.sync_copy(idx_hbm.at[pl.ds(base, B_PER_W)], idx_v)

        def body(c, carry):
            off = pl.multiple_of(c * CHUNK, CHUNK)
            pltpu.async_copy(
                table_hbm.at[idx_v.at[pl.ds(off, CHUNK)]], rows_v, sem
            ).wait()
            pltpu.sync_copy(rows_v, out_hbm.at[pl.ds(base + off, CHUNK)])
            return carry

        lax.fori_loop(0, N_CHUNKS, body, 0)

    return gather_kernel


_gather = _make_gather()


@jax.jit
def kernel(position_ids, PosEnc):
    idx = position_ids.reshape(TOTAL).astype(jnp.int32)
    out = _gather(idx, PosEnc)
    return out.reshape(position_ids.shape + (D,))


# speedup vs baseline: 2.3860x; 1.1998x over previous
import functools

import jax
import jax.numpy as jnp
from jax import lax
from jax.experimental import pallas as pl
from jax.experimental.pallas import tpu as pltpu
from jax.experimental.pallas import tpu_sc as plsc

D = 1024
TOTAL = 4 * 8192
NW = 32
B_PER_W = TOTAL // NW
CHUNK = 16
NBUF = 4
N_CHUNKS = B_PER_W // CHUNK
N_OUTER = N_CHUNKS // NBUF


def _make_gather():
    mesh = plsc.VectorSubcoreMesh(core_axis_name="c", subcore_axis_name="s")

    @functools.partial(
        pl.kernel,
        mesh=mesh,
        out_type=jax.ShapeDtypeStruct((TOTAL, D), jnp.float32),
        scratch_types=[
            pltpu.VMEM((B_PER_W,), jnp.int32),
            pltpu.VMEM((NBUF, CHUNK, D), jnp.float32),
            pltpu.SemaphoreType.DMA((NBUF,)),
            pltpu.SemaphoreType.DMA((NBUF,)),
        ],
    )
    def gather_kernel(idx_hbm, table_hbm, out_hbm, idx_v, rows_v, gsem, osem):
        wid = lax.axis_index("s") * 2 + lax.axis_index("c")
        base = wid * B_PER_W
        pltpu.sync_copy(idx_hbm.at[pl.ds(base, B_PER_W)], idx_v)

        def gather_chunk(c, b):
            off = pl.multiple_of(c * CHUNK, CHUNK)
            return pltpu.make_async_copy(
                table_hbm.at[idx_v.at[pl.ds(off, CHUNK)]],
                rows_v.at[b],
                gsem.at[b],
            )

        def out_chunk(c, b):
            off = pl.multiple_of(c * CHUNK, CHUNK)
            return pltpu.make_async_copy(
                rows_v.at[b],
                out_hbm.at[pl.ds(base + off, CHUNK)],
                osem.at[b],
            )

        for b in range(NBUF):
            gather_chunk(b, b).start()

        def body(g, carry):
            for b in range(NBUF):
                c = g * NBUF + b
                gather_chunk(c, b).wait()
                out_chunk(c, b).start()
                out_chunk(c, b).wait()

                @pl.when(g < N_OUTER - 1)
                def _():
                    gather_chunk(c + NBUF, b).start()

            return carry

        lax.fori_loop(0, N_OUTER, body, 0)

    return gather_kernel


_gather = _make_gather()


@jax.jit
def kernel(position_ids, PosEnc):
    idx = position_ids.reshape(TOTAL).astype(jnp.int32)
    out = _gather(idx, PosEnc)
    return out.reshape(position_ids.shape + (D,))
